# baseline (speedup 1.0000x reference)
import functools
import math

import jax
import jax.numpy as jnp
import numpy as np
from jax import lax
from jax.experimental import pallas as pl
from jax.experimental.pallas import tpu as pltpu
from jax.experimental.pallas import tpu_sc as plsc

N = 512
NIMG = 32
NBINS = 368
NCORES = 2
NSUB = 16
IMGS_PER_CORE = NIMG // NCORES
HROWS = 264
NPIX = HROWS * N
SLAB = NPIX // NSUB
GROUPS = SLAB // 16


def _build_constants():
    eye = np.eye(N)
    v = np.concatenate([eye[:, ::2], eye[:, 1::2][:, ::-1]], axis=1)
    Vc = np.fft.fft(v, axis=1)
    kk = np.arange(N) * (np.pi / (2.0 * N))
    D = (2.0 * (Vc.real * np.cos(kk) - Vc.imag * np.sin(kk))).T
    W = np.exp(-2j * np.pi * np.outer(np.arange(N), np.arange(N)) / N)
    A = W @ D

    ah = np.zeros((HROWS, N), np.complex128)
    ah[: N // 2 + 1] = A[: N // 2 + 1]
    k2 = ah.real.astype(np.float32)
    k3 = ah.imag.astype(np.float32)
    art = A.real.T[:, : N // 2].astype(np.float32)
    ait = A.imag.T[:, : N // 2].copy()
    ait[:, 0] = A[N // 2, :].real
    ait = ait.astype(np.float32)

    f = np.arange(N)
    f = np.where(f < N // 2, f, f - N).astype(np.float64)
    fu = f[: N // 2 + 1]
    colf = np.concatenate([np.arange(N // 2 + 1), np.arange(1, N // 2)])
    radh = np.full((HROWS, N), NBINS - 1, np.int32)
    radh[: N // 2 + 1] = np.sqrt(
        fu[:, None] ** 2 + colf[None, :].astype(np.float64) ** 2
    ).astype(np.int32)
    rad = np.sqrt(f[None, :] ** 2 + f[:, None] ** 2).astype(np.int32)
    nr = np.bincount(rad.ravel(), minlength=NBINS).astype(np.float32)
    return k2, k3, art, ait, radh.ravel(), nr


_K2, _K3, _ART, _AIT, _RAD, _NR = _build_constants()


def _mag_body(x_ref, arth_ref, artl_ref, aith_ref, aitl_ref,
              k2h_ref, k2l_ref, k3h_ref, k3l_ref, out_ref):
    g = (0.2989 * x_ref[0, 0] + 0.587 * x_ref[0, 1] + 0.114 * x_ref[0, 2])
    dot = functools.partial(
        lax.dot_general,
        dimension_numbers=(((1,), (0,)), ((), ())),
        precision=jax.lax.Precision.HIGHEST,
        preferred_element_type=jnp.float32,
    )
    yr = dot(g, arth_ref[...]) + dot(g, artl_ref[...])
    yi = dot(g, aith_ref[...]) + dot(g, aitl_ref[...])
    c0 = lax.broadcasted_iota(jnp.int32, (N, N // 2), 1) == 0
    yi_l = jnp.where(c0, 0.0, yi)
    nyq = yi[:, 0:1]

    U = dot(k2h_ref[...], yr) + dot(k2l_ref[...], yr)
    T = dot(k3h_ref[...], yr) + dot(k3l_ref[...], yr)
    S = dot(k2h_ref[...], yi_l) + dot(k2l_ref[...], yi_l)
    V = dot(k3h_ref[...], yi_l) + dot(k3l_ref[...], yi_l)
    nyq_row = lax.transpose(nyq, (1, 0))
    fr = jnp.sum((k2h_ref[...] + k2l_ref[...]) * nyq_row, axis=1,
                 keepdims=True)
    fi = jnp.sum((k3h_ref[...] + k3l_ref[...]) * nyq_row, axis=1,
                 keepdims=True)

    c0h = lax.broadcasted_iota(jnp.int32, (HROWS, N // 2), 1) == 0

    def lmag(re, im):
        re = re + 1e-8
        im = im + 1e-8
        return jnp.log(jnp.sqrt(re * re + im * im + 1e-10) + 1e-10)

    mag_l = lmag(U - V, S + T)
    mag_g = lmag(jnp.where(c0h, fr, U + V), jnp.where(c0h, fi, S - T))
    row = lax.broadcasted_iota(jnp.int32, (HROWS, N // 2), 0)
    w = jnp.where(row >= N // 2 + 1, 0.0,
                  jnp.where((row == 0) | (row == N // 2), 1.0, 2.0))
    out_ref[0, :, : N // 2] = w * mag_l
    out_ref[0, :, N // 2:] = w * mag_g


def _split_hi_lo(m):
    hi = m.astype(jnp.bfloat16).astype(jnp.float32)
    return hi, m - hi


def _mag_stage(x, art, ait, k2, k3):
    mats = []
    specs = []
    stage1_spec = pl.BlockSpec((N, N // 2), lambda i: (0, 0))
    half_spec = pl.BlockSpec((HROWS, N), lambda i: (0, 0))
    for m in (art, ait):
        mats.extend(_split_hi_lo(m))
        specs.extend([stage1_spec, stage1_spec])
    for m in (k2, k3):
        mats.extend(_split_hi_lo(m))
        specs.extend([half_spec, half_spec])
    nimg = x.shape[0]
    return pl.pallas_call(
        _mag_body,
        grid=(nimg,),
        in_specs=[pl.BlockSpec((1, 3, N, N), lambda i: (i, 0, 0, 0))] + specs,
        out_specs=pl.BlockSpec((1, HROWS, N), lambda i: (i, 0, 0)),
        out_shape=jax.ShapeDtypeStruct((nimg, HROWS, N), jnp.float32),
    )(x, *mats)


@functools.cache
def _hist_kernel_fn(n_img):
    ipc = n_img // NCORES
    mesh = plsc.VectorSubcoreMesh(core_axis_name="c", subcore_axis_name="s")
    return pl.kernel(
        functools.partial(_hist_body, ipc),
        mesh=mesh,
        out_type=jax.ShapeDtypeStruct((n_img, 16), jnp.float32),
        compiler_params=pltpu.CompilerParams(needs_layout_passes=False),
        scratch_types=[
            pltpu.VMEM((SLAB,), jnp.int32),
            pltpu.VMEM((SLAB,), jnp.float32),
            pltpu.VMEM((ipc * NBINS,), jnp.float32),
            pltpu.VMEM((ipc * NBINS,), jnp.float32),
            pltpu.VMEM((NBINS,), jnp.float32),
            pltpu.VMEM((NBINS,), jnp.float32),
            pltpu.VMEM((16,), jnp.float32),
            pltpu.VMEM((NBINS,), jnp.float32),
            pltpu.VMEM((NBINS,), jnp.float32),
            pltpu.VMEM((16,), jnp.float32),
            pltpu.VMEM_SHARED((NSUB * ipc * NBINS,), jnp.float32),
        ],
    )


def _hist_body(ipc, mag_hbm, rad_hbm, nr_hbm, w_hbm, blin_hbm, out_hbm,
               idx_v, vals_v, bins_v, bins_b, nr_v, w_v, blin_v,
               prof_v, tmp_v, orow_v, shared):
    c = lax.axis_index("c")
    s = lax.axis_index("s")
    simg = s % ipc

    pltpu.sync_copy(rad_hbm.at[pl.ds(s * SLAB, SLAB)], idx_v)
    pltpu.sync_copy(nr_hbm, nr_v)
    pltpu.sync_copy(w_hbm, w_v)
    pltpu.sync_copy(blin_hbm, blin_v)

    zero16 = jnp.zeros((16,), jnp.float32)

    def _zero_grp(j, _):
        bins_v[pl.ds(j * 16, 16)] = zero16
        bins_b[pl.ds(j * 16, 16)] = zero16
        return 0

    lax.fori_loop(0, ipc * NBINS // 16, _zero_grp, 0)

    def _img(i, _):
        pltpu.sync_copy(
            mag_hbm.at[c * ipc + i, pl.ds(s * SLAB, SLAB)], vals_v)
        base = i * NBINS

        def _grp(j, _):
            v = vals_v[pl.ds(j * 32, 16)]
            ix = idx_v[pl.ds(j * 32, 16)] + base
            plsc.addupdate_scatter(bins_v, [ix], v)
            v2 = vals_v[pl.ds(j * 32 + 16, 16)]
            ix2 = idx_v[pl.ds(j * 32 + 16, 16)] + base
            plsc.addupdate_scatter(bins_b, [ix2], v2)
            return 0

        return lax.fori_loop(0, GROUPS // 2, _grp, 0)

    lax.fori_loop(0, ipc, _img, 0)

    def _merge(j, _):
        bins_v[pl.ds(j * 16, 16)] = (
            bins_v[pl.ds(j * 16, 16)] + bins_b[pl.ds(j * 16, 16)])
        return 0

    lax.fori_loop(0, ipc * NBINS // 16, _merge, 0)

    pltpu.sync_copy(bins_v, shared.at[pl.ds(s * ipc * NBINS, ipc * NBINS)])
    plsc.subcore_barrier()

    def _zero_prof(j, _):
        prof_v[pl.ds(j * 16, 16)] = zero16
        return 0

    lax.fori_loop(0, NBINS // 16, _zero_prof, 0)

    def _red(w, _):
        pltpu.sync_copy(
            shared.at[pl.ds(w * (ipc * NBINS) + simg * NBINS, NBINS)],
            tmp_v)

        def _acc(g, _):
            prof_v[pl.ds(g * 16, 16)] = (
                prof_v[pl.ds(g * 16, 16)] + tmp_v[pl.ds(g * 16, 16)])
            return 0

        return lax.fori_loop(0, NBINS // 16, _acc, 0)

    lax.fori_loop(0, NSUB, _red, 0)

    big = jnp.float32(3e38)
    lanes = lax.iota(jnp.int32, 16)

    def _minmax(g, carry):
        mn_a, mx_a = carry
        p = prof_v[pl.ds(g * 16, 16)] / (nr_v[pl.ds(g * 16, 16)] + 1e-10)
        gi = lanes + g * 16
        m = (gi >= 1) & (gi <= 360)
        mn_a = jnp.minimum(mn_a, jnp.where(m, p, big))
        mx_a = jnp.maximum(mx_a, jnp.where(m, p, -big))
        return mn_a, mx_a

    mn_a, mx_a = lax.fori_loop(
        0, NBINS // 16, _minmax,
        (jnp.full((16,), big, jnp.float32), jnp.full((16,), -big, jnp.float32)))
    mn = jnp.min(mn_a)
    mx = jnp.max(mx_a)

    def _dot(g, acc):
        p = prof_v[pl.ds(g * 16, 16)] / (nr_v[pl.ds(g * 16, 16)] + 1e-10)
        q = (p - mn) / (mx - mn)
        q = jnp.where(q != q, 0.0, q)
        return acc + q * w_v[pl.ds(g * 16, 16)]

    acc = lax.fori_loop(0, NBINS // 16, _dot, zero16)
    total = jnp.sum(acc)
    orow_v[...] = jnp.where(lanes == 0, total + blin_v[...], 0.0)
    pltpu.sync_copy(orow_v, out_hbm.at[c * ipc + simg])


def kernel(input, Wlin, blin):
    mats = (jnp.asarray(_ART), jnp.asarray(_AIT),
            jnp.asarray(_K2), jnp.asarray(_K3))
    rad = jnp.asarray(_RAD)
    nr = jnp.asarray(_NR)
    wpad = jnp.zeros((NBINS,), jnp.float32).at[180:361].set(Wlin[0])
    blin16 = jnp.zeros((16,), jnp.float32).at[0].set(blin[0])

    mag = _mag_stage(input, *mats)
    out = _hist_kernel_fn(NIMG)(mag.reshape(NIMG, NPIX), rad, nr, wpad,
                                blin16)
    return out[:, :1]


# speedup vs baseline: 1.0113x; 1.0113x over previous
import functools
import math

import jax
import jax.numpy as jnp
import numpy as np
from jax import lax
from jax.experimental import pallas as pl
from jax.experimental.pallas import tpu as pltpu
from jax.experimental.pallas import tpu_sc as plsc

N = 512
NIMG = 32
NBINS = 368
NCORES = 2
NSUB = 16
IMGS_PER_CORE = NIMG // NCORES
HROWS = 264
NPIX = HROWS * N
SLAB = NPIX // NSUB
GROUPS = SLAB // 16


def _build_constants():
    eye = np.eye(N)
    v = np.concatenate([eye[:, ::2], eye[:, 1::2][:, ::-1]], axis=1)
    Vc = np.fft.fft(v, axis=1)
    kk = np.arange(N) * (np.pi / (2.0 * N))
    D = (2.0 * (Vc.real * np.cos(kk) - Vc.imag * np.sin(kk))).T
    W = np.exp(-2j * np.pi * np.outer(np.arange(N), np.arange(N)) / N)
    A = W @ D

    ah = np.zeros((HROWS, N), np.complex128)
    ah[: N // 2 + 1] = A[: N // 2 + 1]
    k2 = ah.real.astype(np.float32)
    k3 = ah.imag.astype(np.float32)
    art = A.real.T[:, : N // 2].astype(np.float32)
    ait = A.imag.T[:, : N // 2].copy()
    ait[:, 0] = A[N // 2, :].real
    ait = ait.astype(np.float32)

    f = np.arange(N)
    f = np.where(f < N // 2, f, f - N).astype(np.float64)
    fu = f[: N // 2 + 1]
    colf = np.concatenate([np.arange(N // 2 + 1), np.arange(1, N // 2)])
    radh = np.full((HROWS, N), NBINS - 1, np.int32)
    radh[: N // 2 + 1] = np.sqrt(
        fu[:, None] ** 2 + colf[None, :].astype(np.float64) ** 2
    ).astype(np.int32)
    rad = np.sqrt(f[None, :] ** 2 + f[:, None] ** 2).astype(np.int32)
    nr = np.bincount(rad.ravel(), minlength=NBINS).astype(np.float32)
    return k2, k3, art, ait, radh.ravel(), nr


_K2, _K3, _ART, _AIT, _RAD, _NR = _build_constants()


def _mag_body(x_ref, arth_ref, artl_ref, aith_ref, aitl_ref,
              k2h_ref, k2l_ref, k3h_ref, k3l_ref, out_ref):
    g = (0.2989 * x_ref[0, 0] + 0.587 * x_ref[0, 1] + 0.114 * x_ref[0, 2])
    dot = functools.partial(
        lax.dot_general,
        dimension_numbers=(((1,), (0,)), ((), ())),
        precision=jax.lax.Precision.HIGHEST,
        preferred_element_type=jnp.float32,
    )
    yr = dot(g, arth_ref[...]) + dot(g, artl_ref[...])
    yi = dot(g, aith_ref[...]) + dot(g, aitl_ref[...])
    c0 = lax.broadcasted_iota(jnp.int32, (N, N // 2), 1) == 0
    yi_l = jnp.where(c0, 0.0, yi)
    nyq = yi[:, 0:1]

    U = dot(k2h_ref[...], yr) + dot(k2l_ref[...], yr)
    T = dot(k3h_ref[...], yr) + dot(k3l_ref[...], yr)
    S = dot(k2h_ref[...], yi_l) + dot(k2l_ref[...], yi_l)
    V = dot(k3h_ref[...], yi_l) + dot(k3l_ref[...], yi_l)
    nyq_row = lax.transpose(nyq, (1, 0))
    fr = jnp.sum((k2h_ref[...] + k2l_ref[...]) * nyq_row, axis=1,
                 keepdims=True)
    fi = jnp.sum((k3h_ref[...] + k3l_ref[...]) * nyq_row, axis=1,
                 keepdims=True)

    c0h = lax.broadcasted_iota(jnp.int32, (HROWS, N // 2), 1) == 0

    def lmag(re, im):
        re = re + 1e-8
        im = im + 1e-8
        return jnp.log(jnp.sqrt(re * re + im * im + 1e-10) + 1e-10)

    mag_l = lmag(U - V, S + T)
    mag_g = lmag(jnp.where(c0h, fr, U + V), jnp.where(c0h, fi, S - T))
    row = lax.broadcasted_iota(jnp.int32, (HROWS, N // 2), 0)
    w = jnp.where(row >= N // 2 + 1, 0.0,
                  jnp.where((row == 0) | (row == N // 2), 1.0, 2.0))
    out_ref[0, :, : N // 2] = w * mag_l
    out_ref[0, :, N // 2:] = w * mag_g


def _split_hi_lo(m):
    hi = m.astype(jnp.bfloat16).astype(jnp.float32)
    return hi, m - hi


def _mag_stage(x, art, ait, k2, k3):
    mats = []
    specs = []
    stage1_spec = pl.BlockSpec((N, N // 2), lambda i: (0, 0))
    half_spec = pl.BlockSpec((HROWS, N), lambda i: (0, 0))
    for m in (art, ait):
        mats.extend(_split_hi_lo(m))
        specs.extend([stage1_spec, stage1_spec])
    for m in (k2, k3):
        mats.extend(_split_hi_lo(m))
        specs.extend([half_spec, half_spec])
    nimg = x.shape[0]
    return pl.pallas_call(
        _mag_body,
        grid=(nimg,),
        in_specs=[pl.BlockSpec((1, 3, N, N), lambda i: (i, 0, 0, 0))] + specs,
        out_specs=pl.BlockSpec((1, HROWS, N), lambda i: (i, 0, 0)),
        out_shape=jax.ShapeDtypeStruct((nimg, HROWS, N), jnp.float32),
    )(x, *mats)


@functools.cache
def _hist_kernel_fn(n_img):
    ipc = n_img // NCORES
    mesh = plsc.VectorSubcoreMesh(core_axis_name="c", subcore_axis_name="s")
    return pl.kernel(
        functools.partial(_hist_body, ipc),
        mesh=mesh,
        out_type=jax.ShapeDtypeStruct((n_img, 16), jnp.float32),
        compiler_params=pltpu.CompilerParams(needs_layout_passes=False),
        scratch_types=[
            pltpu.VMEM((SLAB,), jnp.int32),
            pltpu.VMEM((SLAB,), jnp.float32),
            pltpu.VMEM((ipc * NBINS,), jnp.float32),
            pltpu.VMEM((NBINS,), jnp.float32),
            pltpu.VMEM((NBINS,), jnp.float32),
            pltpu.VMEM((16,), jnp.float32),
            pltpu.VMEM((NBINS,), jnp.float32),
            pltpu.VMEM((NBINS,), jnp.float32),
            pltpu.VMEM((16,), jnp.float32),
            pltpu.VMEM_SHARED((NSUB * ipc * NBINS,), jnp.float32),
        ],
    )


def _hist_body(ipc, mag_hbm, rad_hbm, nr_hbm, w_hbm, blin_hbm, out_hbm,
               idx_v, vals_v, bins_v, nr_v, w_v, blin_v,
               prof_v, tmp_v, orow_v, shared):
    c = lax.axis_index("c")
    s = lax.axis_index("s")
    simg = s % ipc

    pltpu.sync_copy(rad_hbm.at[pl.ds(s * SLAB, SLAB)], idx_v)
    pltpu.sync_copy(nr_hbm, nr_v)
    pltpu.sync_copy(w_hbm, w_v)
    pltpu.sync_copy(blin_hbm, blin_v)

    zero16 = jnp.zeros((16,), jnp.float32)

    def _zero_grp(j, _):
        bins_v[pl.ds(j * 16, 16)] = zero16
        return 0

    lax.fori_loop(0, ipc * NBINS // 16, _zero_grp, 0)

    def _img(i, _):
        pltpu.sync_copy(
            mag_hbm.at[c * ipc + i, pl.ds(s * SLAB, SLAB)], vals_v)
        base = i * NBINS

        def _grp(j, _):
            v = vals_v[pl.ds(j * 16, 16)]
            ix = idx_v[pl.ds(j * 16, 16)] + base
            plsc.addupdate_scatter(bins_v, [ix], v)
            return 0

        return lax.fori_loop(0, GROUPS, _grp, 0)

    lax.fori_loop(0, ipc, _img, 0)

    pltpu.sync_copy(bins_v, shared.at[pl.ds(s * ipc * NBINS, ipc * NBINS)])
    plsc.subcore_barrier()

    def _zero_prof(j, _):
        prof_v[pl.ds(j * 16, 16)] = zero16
        return 0

    lax.fori_loop(0, NBINS // 16, _zero_prof, 0)

    def _red(w, _):
        pltpu.sync_copy(
            shared.at[pl.ds(w * (ipc * NBINS) + simg * NBINS, NBINS)],
            tmp_v)

        def _acc(g, _):
            prof_v[pl.ds(g * 16, 16)] = (
                prof_v[pl.ds(g * 16, 16)] + tmp_v[pl.ds(g * 16, 16)])
            return 0

        return lax.fori_loop(0, NBINS // 16, _acc, 0)

    lax.fori_loop(0, NSUB, _red, 0)

    big = jnp.float32(3e38)
    lanes = lax.iota(jnp.int32, 16)

    def _minmax(g, carry):
        mn_a, mx_a = carry
        p = prof_v[pl.ds(g * 16, 16)] / (nr_v[pl.ds(g * 16, 16)] + 1e-10)
        gi = lanes + g * 16
        m = (gi >= 1) & (gi <= 360)
        mn_a = jnp.minimum(mn_a, jnp.where(m, p, big))
        mx_a = jnp.maximum(mx_a, jnp.where(m, p, -big))
        return mn_a, mx_a

    mn_a, mx_a = lax.fori_loop(
        0, NBINS // 16, _minmax,
        (jnp.full((16,), big, jnp.float32), jnp.full((16,), -big, jnp.float32)))
    mn = jnp.min(mn_a)
    mx = jnp.max(mx_a)

    def _dot(g, acc):
        p = prof_v[pl.ds(g * 16, 16)] / (nr_v[pl.ds(g * 16, 16)] + 1e-10)
        q = (p - mn) / (mx - mn)
        q = jnp.where(q != q, 0.0, q)
        return acc + q * w_v[pl.ds(g * 16, 16)]

    acc = lax.fori_loop(0, NBINS // 16, _dot, zero16)
    total = jnp.sum(acc)
    orow_v[...] = jnp.where(lanes == 0, total + blin_v[...], 0.0)
    pltpu.sync_copy(orow_v, out_hbm.at[c * ipc + simg])


def kernel(input, Wlin, blin):
    mats = (jnp.asarray(_ART), jnp.asarray(_AIT),
            jnp.asarray(_K2), jnp.asarray(_K3))
    rad = jnp.asarray(_RAD)
    nr = jnp.asarray(_NR)
    wpad = jnp.zeros((NBINS,), jnp.float32).at[180:361].set(Wlin[0])
    blin16 = jnp.zeros((16,), jnp.float32).at[0].set(blin[0])

    mag = _mag_stage(input, *mats)
    out = _hist_kernel_fn(NIMG)(mag.reshape(NIMG, NPIX), rad, nr, wpad,
                                blin16)
    return out[:, :1]
